# baseline (speedup 1.0000x reference)
import functools

import jax
import jax.numpy as jnp
from jax.experimental import pallas as pl


def _gcn_kernel(x_ref, adj_ref, w_ref, b_ref, out_ref, *, nk):
    k = pl.program_id(0)
    h = jnp.dot(x_ref[...], w_ref[...], preferred_element_type=jnp.float32)
    contrib = jax.lax.dot_general(
        adj_ref[...].astype(jnp.bfloat16), h.astype(jnp.bfloat16),
        (((0,), (0,)), ((), ())),
        preferred_element_type=jnp.float32)

    @pl.when(k == 0)
    def _():
        out_ref[...] = contrib

    @pl.when(k > 0)
    def _():
        out_ref[...] += contrib

    @pl.when(k == nk - 1)
    def _():
        out_ref[...] = jnp.maximum(out_ref[...] + b_ref[...], 0.0)


def kernel(x, adj, W, b):
    n, d_in = x.shape
    d_out = W.shape[1]

    k_tile = 200
    if n % k_tile:
        k_tile = n
    nk = n // k_tile

    b2 = b.reshape(1, d_out).astype(jnp.float32)

    out = pl.pallas_call(
        functools.partial(_gcn_kernel, nk=nk),
        grid=(nk,),
        in_specs=[
            pl.BlockSpec((k_tile, d_in), lambda k: (k, 0)),
            pl.BlockSpec((k_tile, n), lambda k: (k, 0)),
            pl.BlockSpec((d_in, d_out), lambda k: (0, 0)),
            pl.BlockSpec((1, d_out), lambda k: (0, 0)),
        ],
        out_specs=pl.BlockSpec((n, d_out), lambda k: (0, 0)),
        out_shape=jax.ShapeDtypeStruct((n, d_out), jnp.float32),
    )(x, adj, W, b2)
    return (out, adj)


# speedup vs baseline: 1.0511x; 1.0511x over previous
import functools

import jax
import jax.numpy as jnp
from jax.experimental import pallas as pl


def _gcn_kernel(*refs, nk, nsplit):
    x_ref = refs[0]
    adj_refs = refs[1:1 + nsplit]
    w_ref = refs[1 + nsplit]
    b_ref = refs[2 + nsplit]
    out_ref = refs[3 + nsplit]

    k = pl.program_id(0)
    h = jnp.dot(x_ref[...], w_ref[...],
                preferred_element_type=jnp.float32).astype(jnp.bfloat16)
    sub = x_ref.shape[0] // nsplit
    contrib = None
    for i in range(nsplit):
        c = jax.lax.dot_general(
            adj_refs[i][...].astype(jnp.bfloat16),
            h[i * sub:(i + 1) * sub],
            (((0,), (0,)), ((), ())),
            preferred_element_type=jnp.float32)
        contrib = c if contrib is None else contrib + c

    @pl.when(k == 0)
    def _():
        out_ref[...] = contrib

    @pl.when(k > 0)
    def _():
        out_ref[...] += contrib

    @pl.when(k == nk - 1)
    def _():
        out_ref[...] = jnp.maximum(out_ref[...] + b_ref[...], 0.0)


def kernel(x, adj, W, b):
    n, d_in = x.shape
    d_out = W.shape[1]

    k_tile = 400
    nsplit = 2
    if n % k_tile or (k_tile // nsplit) % 8:
        k_tile, nsplit = n, 1
    nk = n // k_tile
    sub = k_tile // nsplit

    b2 = b.reshape(1, d_out).astype(jnp.float32)

    adj_specs = [
        pl.BlockSpec((sub, n), functools.partial(
            lambda k, i=0: (nsplit * k + i, 0), i=i))
        for i in range(nsplit)
    ]

    out = pl.pallas_call(
        functools.partial(_gcn_kernel, nk=nk, nsplit=nsplit),
        grid=(nk,),
        in_specs=[pl.BlockSpec((k_tile, d_in), lambda k: (k, 0))]
        + adj_specs
        + [
            pl.BlockSpec((d_in, d_out), lambda k: (0, 0)),
            pl.BlockSpec((1, d_out), lambda k: (0, 0)),
        ],
        out_specs=pl.BlockSpec((n, d_out), lambda k: (0, 0)),
        out_shape=jax.ShapeDtypeStruct((n, d_out), jnp.float32),
    )(x, *([adj] * nsplit), W, b2)
    return (out, adj)


# speedup vs baseline: 1.0527x; 1.0015x over previous
import functools

import jax
import jax.numpy as jnp
from jax.experimental import pallas as pl
from jax.experimental.pallas import tpu as pltpu


def _gcn_kernel(x_ref, adj_ref, w_ref, b_ref, out_ref):
    h = jnp.dot(x_ref[...], w_ref[...],
                preferred_element_type=jnp.float32).astype(jnp.bfloat16)
    contrib = jax.lax.dot_general(
        adj_ref[...].astype(jnp.bfloat16), h,
        (((0,), (0,)), ((), ())),
        preferred_element_type=jnp.float32)
    out_ref[...] = jnp.maximum(contrib + b_ref[...], 0.0)


def kernel(x, adj, W, b):
    n, d_in = x.shape
    d_out = W.shape[1]

    d_tile = 512
    nd = pl.cdiv(n, d_tile)

    b2 = b.reshape(1, d_out).astype(jnp.float32)

    out = pl.pallas_call(
        _gcn_kernel,
        grid=(nd,),
        in_specs=[
            pl.BlockSpec((n, d_in), lambda j: (0, 0)),
            pl.BlockSpec((n, d_tile), lambda j: (0, j)),
            pl.BlockSpec((d_in, d_out), lambda j: (0, 0)),
            pl.BlockSpec((1, d_out), lambda j: (0, 0)),
        ],
        out_specs=pl.BlockSpec((d_tile, d_out), lambda j: (j, 0)),
        out_shape=jax.ShapeDtypeStruct((n, d_out), jnp.float32),
        compiler_params=pltpu.CompilerParams(
            dimension_semantics=("parallel",)),
    )(x, adj, W, b2)
    return (out, adj)
